# baseline (speedup 1.0000x reference)
import functools

import jax
import jax.numpy as jnp
from jax import lax
from jax.experimental import pallas as pl
from jax.experimental.pallas import tpu as pltpu
from jax.experimental.pallas import tpu_sc as plsc

_F32 = jnp.float32


@functools.lru_cache(maxsize=None)
def _vmesh():
    return plsc.VectorSubcoreMesh(core_axis_name="c", subcore_axis_name="s")


_NCORES = 2
_NSUB = 16
_WG = 128
_WS = 128


def _mm_body(x_ref, w_ref, o_ref):
    o_ref[...] = jnp.dot(x_ref[...], w_ref[...], preferred_element_type=_F32)


def _tc_matmul(x, w):
    n, d = x.shape
    dout = w.shape[1]
    r = 1024
    return pl.pallas_call(
        _mm_body,
        grid=(n // r,),
        in_specs=[
            pl.BlockSpec((r, d), lambda i: (i, 0)),
            pl.BlockSpec((d, dout), lambda i: (0, 0)),
        ],
        out_specs=pl.BlockSpec((r, dout), lambda i: (i, 0)),
        out_shape=jax.ShapeDtypeStruct((n, dout), _F32),
    )(x, w)


def _tables_body(h_ref, wa_ref, wb_ref, oa_ref, ob_ref):
    h = h_ref[...]
    oa_ref[...] = jnp.dot(h, wa_ref[...], preferred_element_type=_F32)
    ob_ref[...] = jnp.dot(h, wb_ref[...], preferred_element_type=_F32)


def _tc_tables(h, wa, wb):
    n, d = h.shape
    r = 1024
    return pl.pallas_call(
        _tables_body,
        grid=(n // r,),
        in_specs=[
            pl.BlockSpec((r, d), lambda i: (i, 0)),
            pl.BlockSpec((d, d), lambda i: (0, 0)),
            pl.BlockSpec((d, d), lambda i: (0, 0)),
        ],
        out_specs=[
            pl.BlockSpec((r, d), lambda i: (i, 0)),
            pl.BlockSpec((r, d), lambda i: (i, 0)),
        ],
        out_shape=[
            jax.ShapeDtypeStruct((n, d), _F32),
            jax.ShapeDtypeStruct((n, d), _F32),
        ],
    )(h, wa, wb)


def _edge_body(gs_ref, gd_ref, ps_ref, pd_ref, wc_ref, bm1_ref, wm2_ref,
               bm2_ref, o_ref):
    rel = pd_ref[:, 0:3] - ps_ref[:, 0:3]
    d2 = jnp.sum(rel * rel, axis=1, keepdims=True)
    dist = jnp.sqrt(d2 + 1e-12)
    rinv = 1.0 / (dist + 1e-8)
    eaw = (dist * wc_ref[3:4, :]
           + (rel[:, 0:1] * rinv) * wc_ref[0:1, :]
           + (rel[:, 1:2] * rinv) * wc_ref[1:2, :]
           + (rel[:, 2:3] * rinv) * wc_ref[2:3, :])
    m1 = gs_ref[...] + gd_ref[...] + eaw + bm1_ref[...]
    m1 = m1 * jax.nn.sigmoid(m1)
    m2 = jnp.dot(m1, wm2_ref[...], preferred_element_type=_F32) + bm2_ref[...]
    o_ref[...] = m2 * jax.nn.sigmoid(m2)


def _tc_edge(gs, gd, gps, gpd, wc, bm1, wm2, bm2, off=0):
    e, d = gs.shape
    b = 4096
    ob = off // b
    return pl.pallas_call(
        _edge_body,
        grid=(e // b,),
        in_specs=[
            pl.BlockSpec((b, d), lambda i: (i, 0)),
            pl.BlockSpec((b, d), lambda i: (i, 0)),
            pl.BlockSpec((b, d), lambda i, ob=ob: (ob + i, 0)),
            pl.BlockSpec((b, d), lambda i, ob=ob: (ob + i, 0)),
            pl.BlockSpec((4, d), lambda i: (0, 0)),
            pl.BlockSpec((1, d), lambda i: (0, 0)),
            pl.BlockSpec((d, d), lambda i: (0, 0)),
            pl.BlockSpec((1, d), lambda i: (0, 0)),
        ],
        out_specs=pl.BlockSpec((b, d), lambda i: (i, 0)),
        out_shape=jax.ShapeDtypeStruct((e, d), _F32),
    )(gs, gd, gps, gpd, wc, bm1.reshape(1, d), wm2, bm2.reshape(1, d))


def _node_body(h_ref, agg_ref, deg_ref, wu1a_ref, wu1b_ref, bu1_ref, wu2_ref,
               bu2_ref, o_ref):
    degv = jnp.maximum(deg_ref[0, :, 0:1] + deg_ref[1, :, 0:1], 1.0)
    aggm = jnp.sum(agg_ref[...], axis=0) / degv
    h = h_ref[...]
    u = (jnp.dot(h, wu1a_ref[...], preferred_element_type=_F32)
         + jnp.dot(aggm, wu1b_ref[...], preferred_element_type=_F32)
         + bu1_ref[...])
    u = u * jax.nn.sigmoid(u)
    o_ref[...] = h + jnp.dot(u, wu2_ref[...], preferred_element_type=_F32) \
        + bu2_ref[...]


def _tc_node(h, agg2, deg2, wu1, bu1, wu2, bu2):
    n, d = h.shape
    p = agg2.shape[0]
    r = 1024
    return pl.pallas_call(
        _node_body,
        grid=(n // r,),
        in_specs=[
            pl.BlockSpec((r, d), lambda i: (i, 0)),
            pl.BlockSpec((p, r, d), lambda i: (0, i, 0)),
            pl.BlockSpec((2, r, d), lambda i: (0, i, 0)),
            pl.BlockSpec((d, d), lambda i: (0, 0)),
            pl.BlockSpec((d, d), lambda i: (0, 0)),
            pl.BlockSpec((1, d), lambda i: (0, 0)),
            pl.BlockSpec((d, d), lambda i: (0, 0)),
            pl.BlockSpec((1, d), lambda i: (0, 0)),
        ],
        out_specs=pl.BlockSpec((r, d), lambda i: (i, 0)),
        out_shape=jax.ShapeDtypeStruct((n, d), _F32),
    )(h, agg2, deg2, wu1[:d], wu1[d:], bu1.reshape(1, d), wu2,
      bu2.reshape(1, d))


def _readout_body(h_ref, wout_ref, bout_ref, b2_ref, o_ref):
    n = h_ref.shape[0]
    g = o_ref.shape[0]
    s = jnp.dot(h_ref[...], wout_ref[...], preferred_element_type=_F32) \
        + bout_ref[...]
    oh = (b2_ref[...] == lax.broadcasted_iota(jnp.int32, (n, g), 1))
    oh = oh.astype(_F32)
    dn = (((0,), (0,)), ((), ()))
    sums = lax.dot_general(oh, s, dimension_numbers=dn,
                           preferred_element_type=_F32)
    cnt = lax.dot_general(oh, jnp.ones((n, 1), _F32), dimension_numbers=dn,
                          preferred_element_type=_F32)
    o_ref[...] = sums / jnp.maximum(cnt, 1.0)


def _tc_readout(h, w_out, b_out, batch, g):
    n, d = h.shape
    return pl.pallas_call(
        _readout_body,
        grid=(1,),
        in_specs=[
            pl.BlockSpec((n, d), lambda i: (0, 0)),
            pl.BlockSpec((d, 1), lambda i: (0, 0)),
            pl.BlockSpec((1, 1), lambda i: (0, 0)),
            pl.BlockSpec((n, 1), lambda i: (0, 0)),
        ],
        out_specs=pl.BlockSpec((g, 1), lambda i: (0, 0)),
        out_shape=jax.ShapeDtypeStruct((g, 1), _F32),
    )(h, w_out, b_out.reshape(1, 1), batch.reshape(n, 1).astype(jnp.int32))


def _sc_gather2(ta, tb, src, dst, off=0, ne=None):
    n, d = ta.shape
    ne = src.shape[1] if ne is None else ne
    ow = off // _WG

    same = ta is tb
    rows = n // _NSUB

    @functools.partial(
        pl.kernel,
        out_type=(jax.ShapeDtypeStruct((ne, d), _F32),
                  jax.ShapeDtypeStruct((ne, d), _F32)),
        mesh=_vmesh(),
        scratch_types=[pltpu.VMEM_SHARED((n, d), _F32)],
    )
    def k(ta_hbm, tb_hbm, src_hbm, dst_hbm, oa_hbm, ob_hbm, t_sp):
        sid = lax.axis_index("s")
        sl = pl.ds(sid * rows, rows)

        def body(i_v, o_v):
            pltpu.sync_copy(t_sp.at[i_v.at[0]], o_v)

        def run(idx_hbm, out_hbm):
            pltpu.emit_pipeline(
                body,
                grid=(ne // _WG,),
                in_specs=[pl.BlockSpec((1, _WG), lambda i, ow=ow: (0, ow + i))],
                out_specs=[pl.BlockSpec((_WG, d), lambda i: (i, 0))],
                core_axis_name=("c", "s"),
                dimension_semantics=(pltpu.PARALLEL,),
            )(idx_hbm, out_hbm)

        pltpu.sync_copy(ta_hbm.at[sl], t_sp.at[sl])
        plsc.subcore_barrier()
        run(src_hbm, oa_hbm)
        if same:
            run(dst_hbm, ob_hbm)
        else:
            plsc.subcore_barrier()
            pltpu.sync_copy(tb_hbm.at[sl], t_sp.at[sl])
            plsc.subcore_barrier()
            run(dst_hbm, ob_hbm)

    return k(ta, tb, src, dst)


def _sc_scatter(m, dst, n, zeros, off=0):
    e, d = m.shape
    ow = off // _WS
    rows = n // _NSUB

    @functools.partial(
        pl.kernel,
        out_type=jax.ShapeDtypeStruct((_NCORES, n, d), _F32),
        mesh=_vmesh(),
        scratch_types=[pltpu.VMEM_SHARED((n, d), _F32)],
    )
    def k(m_hbm, dst_hbm, z_hbm, o_hbm, acc_sp):
        cid = lax.axis_index("c")
        sid = lax.axis_index("s")
        sl = pl.ds(sid * rows, rows)
        pltpu.sync_copy(z_hbm.at[sl], acc_sp.at[sl])
        plsc.subcore_barrier()

        def body(m_v, i_v):
            pltpu.sync_copy(m_v, acc_sp.at[i_v.at[0]], add=True)

        pltpu.emit_pipeline(
            body,
            grid=(e // _WS,),
            in_specs=[
                pl.BlockSpec((_WS, d), lambda i: (i, 0)),
                pl.BlockSpec((1, _WS), lambda i, ow=ow: (0, ow + i)),
            ],
            out_specs=[],
            core_axis_name=("c", "s"),
            dimension_semantics=(pltpu.PARALLEL,),
        )(m_hbm, dst_hbm)
        plsc.subcore_barrier()

        @pl.loop(0, 4)
        def _wb(c):
            wsl = pl.ds(sid * rows + c * (rows // 4), rows // 4)
            pltpu.sync_copy(acc_sp.at[wsl], o_hbm.at[cid, wsl])

    return k(m, dst, zeros)


def _sc_deg(dst, n, d, zeros):
    e = dst.shape[1]
    rows = n // _NSUB

    @functools.partial(
        pl.kernel,
        out_type=jax.ShapeDtypeStruct((_NCORES, n, d), _F32),
        mesh=_vmesh(),
        scratch_types=[pltpu.VMEM_SHARED((n, d), _F32),
                       pltpu.VMEM((_WS, d), _F32)],
    )
    def k(dst_hbm, z_hbm, o_hbm, deg_sp, ones_v):
        cid = lax.axis_index("c")
        sid = lax.axis_index("s")

        @pl.loop(0, _WS)
        def _fill_row(r):
            @pl.loop(0, d, step=16)
            def _fill_col(c):
                ones_v[r, pl.ds(c, 16)] = jnp.ones((16,), _F32)

        sl = pl.ds(sid * rows, rows)
        pltpu.sync_copy(z_hbm.at[sl], deg_sp.at[sl])
        plsc.subcore_barrier()

        def body(i_v):
            pltpu.sync_copy(ones_v, deg_sp.at[i_v.at[0]], add=True)

        pltpu.emit_pipeline(
            body,
            grid=(e // _WS,),
            in_specs=[pl.BlockSpec((1, _WS), lambda i: (0, i))],
            out_specs=[],
            core_axis_name=("c", "s"),
            dimension_semantics=(pltpu.PARALLEL,),
        )(dst_hbm)
        plsc.subcore_barrier()

        @pl.loop(0, 4)
        def _wb(c):
            wsl = pl.ds(sid * rows + c * (rows // 4), rows // 4)
            pltpu.sync_copy(deg_sp.at[wsl], o_hbm.at[cid, wsl])

    return k(dst, zeros)


def kernel(x, pos, edge_index, batch, W_in, Wm1, bm1, Wm2, bm2, Wu1, bu1,
           Wu2, bu2, W_out, b_out):
    n, d = x.shape
    e = edge_index.shape[1]
    num_layers = Wm1.shape[0]
    g = 64

    n2 = ((n + 2047) // 2048) * 2048
    e2 = ((e + 4095) // 4096) * 4096

    ei = edge_index.astype(jnp.int32)
    pad_ar = jnp.arange(e2 - e, dtype=jnp.int32)
    src = jnp.concatenate([ei[0], pad_ar % n]).reshape(1, e2)
    dst = jnp.concatenate([ei[1], n + pad_ar % (n2 - n)]).reshape(1, e2)

    xp = jnp.pad(x.astype(_F32), ((0, n2 - n), (0, 0)))
    posw = jnp.pad(pos.astype(_F32), ((0, n2 - n), (0, d - pos.shape[1])))
    batchp = jnp.concatenate(
        [batch.astype(jnp.int32), jnp.full((n2 - n,), g + 1, jnp.int32)])
    zeros = jnp.zeros((n2, d), _F32)

    h = _tc_matmul(xp, W_in)
    deg2 = _sc_deg(dst, n2, d, zeros)
    gps, gpd = _sc_gather2(posw, posw, src, dst)

    nchunks = 1
    ec = e2 // nchunks
    for l in range(num_layers):
        wa = Wm1[l, :d]
        wb = Wm1[l, d:2 * d]
        wc = Wm1[l, 2 * d:]
        ts, td = _tc_tables(h, wa, wb)
        parts = []
        for c in range(nchunks):
            off = c * ec
            gs, gd = _sc_gather2(ts, td, src, dst, off=off, ne=ec)
            m = _tc_edge(gs, gd, gps, gpd, wc, bm1[l], Wm2[l], bm2[l],
                         off=off)
            parts.append(_sc_scatter(m, dst, n2, zeros, off=off))
        agg2 = jnp.concatenate(parts, axis=0)
        h = _tc_node(h, agg2, deg2, Wu1[l], bu1[l], Wu2[l], bu2[l])

    return _tc_readout(h, W_out, b_out, batchp, g)


# speedup vs baseline: 1.1799x; 1.1799x over previous
import functools

import jax
import jax.numpy as jnp
from jax import lax
from jax.experimental import pallas as pl
from jax.experimental.pallas import tpu as pltpu
from jax.experimental.pallas import tpu_sc as plsc

_F32 = jnp.float32


@functools.lru_cache(maxsize=None)
def _vmesh():
    return plsc.VectorSubcoreMesh(core_axis_name="c", subcore_axis_name="s")


_NCORES = 2
_NSUB = 16
_WG = 128
_WS = 128


def _mm_body(x_ref, w_ref, o_ref):
    o_ref[...] = jnp.dot(x_ref[...], w_ref[...], preferred_element_type=_F32)


def _tc_matmul(x, w):
    n, d = x.shape
    dout = w.shape[1]
    r = 1024
    return pl.pallas_call(
        _mm_body,
        grid=(n // r,),
        in_specs=[
            pl.BlockSpec((r, d), lambda i: (i, 0)),
            pl.BlockSpec((d, dout), lambda i: (0, 0)),
        ],
        out_specs=pl.BlockSpec((r, dout), lambda i: (i, 0)),
        out_shape=jax.ShapeDtypeStruct((n, dout), _F32),
    )(x, w)


def _tables_body(h_ref, wa_ref, wb_ref, oa_ref, ob_ref):
    h = h_ref[...]
    oa_ref[...] = jnp.dot(h, wa_ref[...], preferred_element_type=_F32)
    ob_ref[...] = jnp.dot(h, wb_ref[...], preferred_element_type=_F32)


def _tc_tables(h, wa, wb):
    n, d = h.shape
    r = 1024
    return pl.pallas_call(
        _tables_body,
        grid=(n // r,),
        in_specs=[
            pl.BlockSpec((r, d), lambda i: (i, 0)),
            pl.BlockSpec((d, d), lambda i: (0, 0)),
            pl.BlockSpec((d, d), lambda i: (0, 0)),
        ],
        out_specs=[
            pl.BlockSpec((r, d), lambda i: (i, 0)),
            pl.BlockSpec((r, d), lambda i: (i, 0)),
        ],
        out_shape=[
            jax.ShapeDtypeStruct((n, d), _F32),
            jax.ShapeDtypeStruct((n, d), _F32),
        ],
    )(h, wa, wb)


def _ea_body(ps_ref, pd_ref, o_ref):
    b = ps_ref.shape[0]
    rel = pd_ref[:, 0:3] - ps_ref[:, 0:3]
    d2 = jnp.sum(rel * rel, axis=1, keepdims=True)
    dist = jnp.sqrt(d2 + 1e-12)
    rinv = 1.0 / (dist + 1e-8)
    ea = jnp.concatenate([rel * rinv, dist, jnp.zeros((b, 4), _F32)],
                         axis=1)
    o_ref[...] = ea.T


def _tc_ea(gps, gpd):
    e, d = gps.shape
    b = 4096
    return pl.pallas_call(
        _ea_body,
        grid=(e // b,),
        in_specs=[
            pl.BlockSpec((b, d), lambda i: (i, 0)),
            pl.BlockSpec((b, d), lambda i: (i, 0)),
        ],
        out_specs=pl.BlockSpec((8, b), lambda i: (0, i)),
        out_shape=jax.ShapeDtypeStruct((8, e), _F32),
    )(gps, gpd)


def _edge_body(gs_ref, gd_ref, ea_ref, wc8_ref, bm1_ref, wm2_ref,
               bm2_ref, o_ref):
    dn = (((0,), (0,)), ((), ()))
    eaw = lax.dot_general(ea_ref[...], wc8_ref[...], dimension_numbers=dn,
                          preferred_element_type=_F32)
    m1 = gs_ref[...] + gd_ref[...] + eaw + bm1_ref[...]
    m1 = m1 * jax.nn.sigmoid(m1)
    m2 = jnp.dot(m1, wm2_ref[...], preferred_element_type=_F32) + bm2_ref[...]
    o_ref[...] = m2 * jax.nn.sigmoid(m2)


def _tc_edge(gs, gd, ea8, wc8, bm1, wm2, bm2, off=0):
    e, d = gs.shape
    b = 4096
    ob = off // b
    return pl.pallas_call(
        _edge_body,
        grid=(e // b,),
        in_specs=[
            pl.BlockSpec((b, d), lambda i: (i, 0)),
            pl.BlockSpec((b, d), lambda i: (i, 0)),
            pl.BlockSpec((8, b), lambda i, ob=ob: (0, ob + i)),
            pl.BlockSpec((8, d), lambda i: (0, 0)),
            pl.BlockSpec((1, d), lambda i: (0, 0)),
            pl.BlockSpec((d, d), lambda i: (0, 0)),
            pl.BlockSpec((1, d), lambda i: (0, 0)),
        ],
        out_specs=pl.BlockSpec((b, d), lambda i: (i, 0)),
        out_shape=jax.ShapeDtypeStruct((e, d), _F32),
    )(gs, gd, ea8, wc8, bm1.reshape(1, d), wm2, bm2.reshape(1, d))


def _node_body(h_ref, agg_ref, deg_ref, wu1a_ref, wu1b_ref, bu1_ref, wu2_ref,
               bu2_ref, o_ref):
    degv = jnp.maximum(deg_ref[0, :, 0:1] + deg_ref[1, :, 0:1], 1.0)
    aggm = jnp.sum(agg_ref[...], axis=0) / degv
    h = h_ref[...]
    u = (jnp.dot(h, wu1a_ref[...], preferred_element_type=_F32)
         + jnp.dot(aggm, wu1b_ref[...], preferred_element_type=_F32)
         + bu1_ref[...])
    u = u * jax.nn.sigmoid(u)
    o_ref[...] = h + jnp.dot(u, wu2_ref[...], preferred_element_type=_F32) \
        + bu2_ref[...]


def _tc_node(h, agg2, deg2, wu1, bu1, wu2, bu2):
    n, d = h.shape
    p = agg2.shape[0]
    r = 1024
    return pl.pallas_call(
        _node_body,
        grid=(n // r,),
        in_specs=[
            pl.BlockSpec((r, d), lambda i: (i, 0)),
            pl.BlockSpec((p, r, d), lambda i: (0, i, 0)),
            pl.BlockSpec((2, r, d), lambda i: (0, i, 0)),
            pl.BlockSpec((d, d), lambda i: (0, 0)),
            pl.BlockSpec((d, d), lambda i: (0, 0)),
            pl.BlockSpec((1, d), lambda i: (0, 0)),
            pl.BlockSpec((d, d), lambda i: (0, 0)),
            pl.BlockSpec((1, d), lambda i: (0, 0)),
        ],
        out_specs=pl.BlockSpec((r, d), lambda i: (i, 0)),
        out_shape=jax.ShapeDtypeStruct((n, d), _F32),
    )(h, agg2, deg2, wu1[:d], wu1[d:], bu1.reshape(1, d), wu2,
      bu2.reshape(1, d))


def _readout_body(h_ref, wout_ref, bout_ref, b2_ref, o_ref):
    n = h_ref.shape[0]
    g = o_ref.shape[0]
    s = jnp.dot(h_ref[...], wout_ref[...], preferred_element_type=_F32) \
        + bout_ref[...]
    oh = (b2_ref[...] == lax.broadcasted_iota(jnp.int32, (n, g), 1))
    oh = oh.astype(_F32)
    dn = (((0,), (0,)), ((), ()))
    sums = lax.dot_general(oh, s, dimension_numbers=dn,
                           preferred_element_type=_F32)
    cnt = lax.dot_general(oh, jnp.ones((n, 1), _F32), dimension_numbers=dn,
                          preferred_element_type=_F32)
    o_ref[...] = sums / jnp.maximum(cnt, 1.0)


def _tc_readout(h, w_out, b_out, batch, g):
    n, d = h.shape
    return pl.pallas_call(
        _readout_body,
        grid=(1,),
        in_specs=[
            pl.BlockSpec((n, d), lambda i: (0, 0)),
            pl.BlockSpec((d, 1), lambda i: (0, 0)),
            pl.BlockSpec((1, 1), lambda i: (0, 0)),
            pl.BlockSpec((n, 1), lambda i: (0, 0)),
        ],
        out_specs=pl.BlockSpec((g, 1), lambda i: (0, 0)),
        out_shape=jax.ShapeDtypeStruct((g, 1), _F32),
    )(h, w_out, b_out.reshape(1, 1), batch.reshape(n, 1).astype(jnp.int32))


def _sc_gather2(ta, tb, src, dst, off=0, ne=None):
    n, d = ta.shape
    ne = src.shape[1] if ne is None else ne
    ow = off // _WG

    same = ta is tb
    rows = n // _NSUB

    @functools.partial(
        pl.kernel,
        out_type=(jax.ShapeDtypeStruct((ne, d), _F32),
                  jax.ShapeDtypeStruct((ne, d), _F32)),
        mesh=_vmesh(),
        scratch_types=[pltpu.VMEM_SHARED((n, d), _F32)],
    )
    def k(ta_hbm, tb_hbm, src_hbm, dst_hbm, oa_hbm, ob_hbm, t_sp):
        sid = lax.axis_index("s")
        sl = pl.ds(sid * rows, rows)

        def body(i_v, o_v):
            pltpu.sync_copy(t_sp.at[i_v.at[0]], o_v)

        def run(idx_hbm, out_hbm):
            pltpu.emit_pipeline(
                body,
                grid=(ne // _WG,),
                in_specs=[pl.BlockSpec((1, _WG), lambda i, ow=ow: (0, ow + i))],
                out_specs=[pl.BlockSpec((_WG, d), lambda i: (i, 0))],
                core_axis_name=("c", "s"),
                dimension_semantics=(pltpu.PARALLEL,),
            )(idx_hbm, out_hbm)

        pltpu.sync_copy(ta_hbm.at[sl], t_sp.at[sl])
        plsc.subcore_barrier()
        run(src_hbm, oa_hbm)
        if same:
            run(dst_hbm, ob_hbm)
        else:
            plsc.subcore_barrier()
            pltpu.sync_copy(tb_hbm.at[sl], t_sp.at[sl])
            plsc.subcore_barrier()
            run(dst_hbm, ob_hbm)

    return k(ta, tb, src, dst)


def _sc_scatter(m, dst, n, zeros, off=0):
    e, d = m.shape
    ow = off // _WS
    rows = n // _NSUB

    @functools.partial(
        pl.kernel,
        out_type=jax.ShapeDtypeStruct((_NCORES, n, d), _F32),
        mesh=_vmesh(),
        scratch_types=[pltpu.VMEM_SHARED((n, d), _F32)],
    )
    def k(m_hbm, dst_hbm, z_hbm, o_hbm, acc_sp):
        cid = lax.axis_index("c")
        sid = lax.axis_index("s")
        sl = pl.ds(sid * rows, rows)
        pltpu.sync_copy(z_hbm.at[sl], acc_sp.at[sl])
        plsc.subcore_barrier()

        def body(m_v, i_v):
            pltpu.sync_copy(m_v, acc_sp.at[i_v.at[0]], add=True)

        pltpu.emit_pipeline(
            body,
            grid=(e // _WS,),
            in_specs=[
                pl.BlockSpec((_WS, d), lambda i: (i, 0)),
                pl.BlockSpec((1, _WS), lambda i, ow=ow: (0, ow + i)),
            ],
            out_specs=[],
            core_axis_name=("c", "s"),
            dimension_semantics=(pltpu.PARALLEL,),
        )(m_hbm, dst_hbm)
        plsc.subcore_barrier()

        @pl.loop(0, 4)
        def _wb(c):
            wsl = pl.ds(sid * rows + c * (rows // 4), rows // 4)
            pltpu.sync_copy(acc_sp.at[wsl], o_hbm.at[cid, wsl])

    return k(m, dst, zeros)


def _sc_deg(dst, n, d, zeros):
    e = dst.shape[1]
    rows = n // _NSUB

    @functools.partial(
        pl.kernel,
        out_type=jax.ShapeDtypeStruct((_NCORES, n, d), _F32),
        mesh=_vmesh(),
        scratch_types=[pltpu.VMEM_SHARED((n, d), _F32),
                       pltpu.VMEM((_WS, d), _F32)],
    )
    def k(dst_hbm, z_hbm, o_hbm, deg_sp, ones_v):
        cid = lax.axis_index("c")
        sid = lax.axis_index("s")

        @pl.loop(0, _WS)
        def _fill_row(r):
            @pl.loop(0, d, step=16)
            def _fill_col(c):
                ones_v[r, pl.ds(c, 16)] = jnp.ones((16,), _F32)

        sl = pl.ds(sid * rows, rows)
        pltpu.sync_copy(z_hbm.at[sl], deg_sp.at[sl])
        plsc.subcore_barrier()

        def body(i_v):
            pltpu.sync_copy(ones_v, deg_sp.at[i_v.at[0]], add=True)

        pltpu.emit_pipeline(
            body,
            grid=(e // _WS,),
            in_specs=[pl.BlockSpec((1, _WS), lambda i: (0, i))],
            out_specs=[],
            core_axis_name=("c", "s"),
            dimension_semantics=(pltpu.PARALLEL,),
        )(dst_hbm)
        plsc.subcore_barrier()

        @pl.loop(0, 4)
        def _wb(c):
            wsl = pl.ds(sid * rows + c * (rows // 4), rows // 4)
            pltpu.sync_copy(deg_sp.at[wsl], o_hbm.at[cid, wsl])

    return k(dst, zeros)


def kernel(x, pos, edge_index, batch, W_in, Wm1, bm1, Wm2, bm2, Wu1, bu1,
           Wu2, bu2, W_out, b_out):
    n, d = x.shape
    e = edge_index.shape[1]
    num_layers = Wm1.shape[0]
    g = 64

    n2 = ((n + 2047) // 2048) * 2048
    e2 = ((e + 4095) // 4096) * 4096

    ei = edge_index.astype(jnp.int32)
    pad_ar = jnp.arange(e2 - e, dtype=jnp.int32)
    src = jnp.concatenate([ei[0], pad_ar % n]).reshape(1, e2)
    dst = jnp.concatenate([ei[1], n + pad_ar % (n2 - n)]).reshape(1, e2)

    xp = jnp.pad(x.astype(_F32), ((0, n2 - n), (0, 0)))
    posw = jnp.pad(pos.astype(_F32), ((0, n2 - n), (0, d - pos.shape[1])))
    batchp = jnp.concatenate(
        [batch.astype(jnp.int32), jnp.full((n2 - n,), g + 1, jnp.int32)])
    zeros = jnp.zeros((n2, d), _F32)

    h = _tc_matmul(xp, W_in)
    deg2 = _sc_deg(dst, n2, d, zeros)
    gps, gpd = _sc_gather2(posw, posw, src, dst)
    ea8 = _tc_ea(gps, gpd)

    nchunks = 1
    ec = e2 // nchunks
    for l in range(num_layers):
        wa = Wm1[l, :d]
        wb = Wm1[l, d:2 * d]
        wc8 = jnp.pad(Wm1[l, 2 * d:], ((0, 4), (0, 0)))
        ts, td = _tc_tables(h, wa, wb)
        parts = []
        for c in range(nchunks):
            off = c * ec
            gs, gd = _sc_gather2(ts, td, src, dst, off=off, ne=ec)
            m = _tc_edge(gs, gd, ea8, wc8, bm1[l], Wm2[l], bm2[l],
                         off=off)
            parts.append(_sc_scatter(m, dst, n2, zeros, off=off))
        agg2 = jnp.concatenate(parts, axis=0)
        h = _tc_node(h, agg2, deg2, Wu1[l], bu1[l], Wu2[l], bu2[l])

    return _tc_readout(h, W_out, b_out, batchp, g)
